# baseline (speedup 1.0000x reference)
import functools

import jax
import jax.numpy as jnp
from jax import lax
from jax.experimental import pallas as pl
from jax.experimental.pallas import tpu as pltpu
from jax.experimental.pallas import tpu_sc as plsc

NUM_NODE = 2048
RANK = 64
K_KEEP = int(0.01 * NUM_NODE * NUM_NODE)
NEG = -9000000000000000.0
_ROWS_PER_BLK = 128

_NW = 32
_NELEM = NUM_NODE * NUM_NODE
_EPW = _NELEM // _NW
_CHUNK = 8192
_NCHUNK = _EPW // _CHUNK
_H1BINS = 32768
_H2BINS = 65536
_HIGHEST = jax.lax.Precision.HIGHEST


def _adj_body(m1_ref, m2_ref, out_ref):
    out_ref[...] = jnp.maximum(
        lax.dot_general(
            m1_ref[...], m2_ref[...],
            dimension_numbers=(((1,), (1,)), ((), ())),
            preferred_element_type=jnp.float32,
        ),
        0.0,
    )


def _tc_adj(M1, M2):
    return pl.pallas_call(
        _adj_body,
        grid=(NUM_NODE // _ROWS_PER_BLK,),
        in_specs=[
            pl.BlockSpec((_ROWS_PER_BLK, RANK), lambda i: (i, 0)),
            pl.BlockSpec((NUM_NODE, RANK), lambda i: (0, 0)),
        ],
        out_specs=pl.BlockSpec((_ROWS_PER_BLK, NUM_NODE), lambda i: (i, 0)),
        out_shape=jax.ShapeDtypeStruct((NUM_NODE, NUM_NODE), jnp.float32),
    )(M1, M2)


_SC_MESH = plsc.VectorSubcoreMesh(core_axis_name="c", subcore_axis_name="s")


def _zero_hist(hist, nbins):
    z = jnp.zeros((16,), jnp.float32)

    def zbody(j, _):
        hist[pl.ds(j * 16, 16)] = z
        return 0

    lax.fori_loop(0, nbins // 16, zbody, 0, unroll=8)


@functools.partial(
    pl.kernel,
    out_type=jax.ShapeDtypeStruct((_NW * _H1BINS,), jnp.float32),
    mesh=_SC_MESH,
    compiler_params=pltpu.CompilerParams(needs_layout_passes=False),
    scratch_types=[
        pltpu.VMEM((_CHUNK,), jnp.int32),
        pltpu.VMEM((_H1BINS,), jnp.float32),
    ],
)
def _sc_hist1(adj_hbm, out_hbm, buf, hist):
    wid = lax.axis_index("s") * 2 + lax.axis_index("c")
    base = wid * _EPW
    _zero_hist(hist, _H1BINS)
    ones = jnp.ones((16,), jnp.float32)

    def chunk_body(g, _):
        pltpu.sync_copy(adj_hbm.at[pl.ds(base + g * _CHUNK, _CHUNK)], buf)

        def ebody(j, _):
            b = buf[pl.ds(j * 16, 16)]
            bin1 = lax.shift_right_logical(b, 16)
            plsc.addupdate_scatter(hist, [bin1], ones)
            return 0

        lax.fori_loop(0, _CHUNK // 16, ebody, 0, unroll=8)
        return 0

    lax.fori_loop(0, _NCHUNK, chunk_body, 0)
    pltpu.sync_copy(hist, out_hbm.at[pl.ds(wid * _H1BINS, _H1BINS)])


@functools.partial(
    pl.kernel,
    out_type=jax.ShapeDtypeStruct((_NW * _H2BINS,), jnp.float32),
    mesh=_SC_MESH,
    compiler_params=pltpu.CompilerParams(needs_layout_passes=False),
    scratch_types=[
        pltpu.VMEM((_CHUNK,), jnp.int32),
        pltpu.VMEM((_H2BINS,), jnp.float32),
        pltpu.VMEM((16,), jnp.int32),
    ],
)
def _sc_hist2(adj_hbm, p_hbm, out_hbm, buf, hist, pbuf):
    wid = lax.axis_index("s") * 2 + lax.axis_index("c")
    base = wid * _EPW
    _zero_hist(hist, _H2BINS)
    pltpu.sync_copy(p_hbm.at[pl.ds(0, 16)], pbuf)
    pv = pbuf[...]
    ones = jnp.ones((16,), jnp.float32)
    low = jnp.full((16,), 0xFFFF, jnp.int32)

    def chunk_body(g, _):
        pltpu.sync_copy(adj_hbm.at[pl.ds(base + g * _CHUNK, _CHUNK)], buf)

        def ebody(j, _):
            b = buf[pl.ds(j * 16, 16)]
            hi = lax.shift_right_logical(b, 16)
            bin2 = jnp.bitwise_and(b, low)
            plsc.addupdate_scatter(hist, [bin2], ones, mask=hi == pv)
            return 0

        lax.fori_loop(0, _CHUNK // 16, ebody, 0, unroll=8)
        return 0

    lax.fori_loop(0, _NCHUNK, chunk_body, 0)
    pltpu.sync_copy(hist, out_hbm.at[pl.ds(wid * _H2BINS, _H2BINS)])


def _suffix_counts(merged_f, nrows, ncols):
    rowtot = jnp.sum(merged_f, axis=1)
    a_r = lax.broadcasted_iota(jnp.int32, (nrows, nrows), 0)
    a_c = lax.broadcasted_iota(jnp.int32, (nrows, nrows), 1)
    strict_upper = (a_c > a_r).astype(jnp.float32)
    tail = lax.dot_general(
        strict_upper, rowtot, (((1,), (0,)), ((), ())), precision=_HIGHEST
    )
    u_k = lax.broadcasted_iota(jnp.int32, (ncols, ncols), 0)
    u_c = lax.broadcasted_iota(jnp.int32, (ncols, ncols), 1)
    upper = (u_k >= u_c).astype(jnp.float32)
    colsuf = lax.dot_general(
        merged_f, upper, (((1,), (0,)), ((), ())), precision=_HIGHEST
    )
    return tail[:, None] + colsuf


def _locate1_body(h1_ref, p_ref, k_ref):
    merged = jnp.sum(h1_ref[...], axis=0)
    s = _suffix_counts(merged, 256, 128)
    cond = (s >= float(K_KEEP)).astype(jnp.int32)
    bstar = jnp.sum(cond) - 1
    flatidx = (
        lax.broadcasted_iota(jnp.int32, (256, 128), 0) * 128
        + lax.broadcasted_iota(jnp.int32, (256, 128), 1)
    )
    s_above = jnp.sum(jnp.where(flatidx > bstar, merged, 0.0))
    krem = K_KEEP - s_above.astype(jnp.int32)
    p_ref[...] = jnp.zeros((8, 128), jnp.int32) + bstar
    k_ref[...] = jnp.zeros((8, 128), jnp.int32) + krem


def _tc_locate1(h1r):
    return pl.pallas_call(
        _locate1_body,
        out_shape=[
            jax.ShapeDtypeStruct((8, 128), jnp.int32),
            jax.ShapeDtypeStruct((8, 128), jnp.int32),
        ],
    )(h1r)


def _final_body(adj_ref, h2_ref, p_ref, k_ref, out_ref):
    p_hi = p_ref[0, 0]
    krem = k_ref[0, 0]
    merged = jnp.sum(h2_ref[...], axis=0)
    s = _suffix_counts(merged, 512, 128)
    cond = (s >= krem.astype(jnp.float32)).astype(jnp.int32)
    b2star = jnp.sum(cond) - 1
    kth_bits = jnp.bitwise_or(lax.shift_left(p_hi, 16), b2star)

    adj = adj_ref[...]
    for i in range(NUM_NODE // _ROWS_PER_BLK):
        blk = adj[i * _ROWS_PER_BLK:(i + 1) * _ROWS_PER_BLK, :]
        bbits = lax.bitcast_convert_type(blk, jnp.int32)
        masked = jnp.where(bbits > kth_bits, blk, NEG)
        rowmax = jnp.max(masked, axis=1, keepdims=True)
        e = jnp.exp(masked - rowmax)
        p = e / jnp.sum(e, axis=1, keepdims=True)
        cols = lax.broadcasted_iota(jnp.int32, (_ROWS_PER_BLK, NUM_NODE), 1)
        rows = lax.broadcasted_iota(jnp.int32, (_ROWS_PER_BLK, NUM_NODE), 0)
        rows = rows + i * _ROWS_PER_BLK
        out_ref[i * _ROWS_PER_BLK:(i + 1) * _ROWS_PER_BLK, :] = jnp.where(
            rows == cols, 1.0, p
        )


def _tc_final(adj, h2r, p8, k8):
    return pl.pallas_call(
        _final_body,
        out_shape=jax.ShapeDtypeStruct((NUM_NODE, NUM_NODE), jnp.float32),
    )(adj, h2r, p8, k8)


def kernel(x, M1, M2):
    del x
    adj = _tc_adj(M1, M2)
    flat = lax.bitcast_convert_type(adj, jnp.int32).reshape(-1)
    h1 = _sc_hist1(flat).reshape(_NW, 256, 128)
    p8, k8 = _tc_locate1(h1)
    h2 = _sc_hist2(flat, p8.reshape(-1)).reshape(_NW, 512, 128)
    return _tc_final(adj, h2, p8, k8)


# speedup vs baseline: 1.1463x; 1.1463x over previous
import functools

import jax
import jax.numpy as jnp
from jax import lax
from jax.experimental import pallas as pl
from jax.experimental.pallas import tpu as pltpu
from jax.experimental.pallas import tpu_sc as plsc

NUM_NODE = 2048
RANK = 64
K_KEEP = int(0.01 * NUM_NODE * NUM_NODE)
_ROWS_PER_BLK = 128
_NBLK = NUM_NODE // _ROWS_PER_BLK

_NW = 32
_NELEM = NUM_NODE * NUM_NODE
_EPW = _NELEM // _NW
_CHUNK = 8192
_NCHUNK = _EPW // _CHUNK
_NCOPIES = 4
_HBITS = 14
_HBINS = 1 << _HBITS
_LOWBITS = 31 - _HBITS
_HIGHEST = jax.lax.Precision.HIGHEST


def _adj_body(m1_ref, m2_ref, adj_ref, e_ref):
    a = jnp.maximum(
        lax.dot_general(
            m1_ref[...], m2_ref[...],
            dimension_numbers=(((1,), (1,)), ((), ())),
            preferred_element_type=jnp.float32,
        ),
        0.0,
    )
    adj_ref[...] = a
    rm = jnp.max(a, axis=1, keepdims=True)
    e_ref[...] = jnp.exp(a - rm)


def _tc_adj(M1, M2):
    return pl.pallas_call(
        _adj_body,
        grid=(_NBLK,),
        in_specs=[
            pl.BlockSpec((_ROWS_PER_BLK, RANK), lambda i: (i, 0)),
            pl.BlockSpec((NUM_NODE, RANK), lambda i: (0, 0)),
        ],
        out_specs=[
            pl.BlockSpec((_ROWS_PER_BLK, NUM_NODE), lambda i: (i, 0)),
            pl.BlockSpec((_ROWS_PER_BLK, NUM_NODE), lambda i: (i, 0)),
        ],
        out_shape=[
            jax.ShapeDtypeStruct((NUM_NODE, NUM_NODE), jnp.float32),
            jax.ShapeDtypeStruct((NUM_NODE, NUM_NODE), jnp.float32),
        ],
    )(M1, M2)


_SC_MESH = plsc.VectorSubcoreMesh(core_axis_name="c", subcore_axis_name="s")


def _zero_hist(hist, nbins):
    z = jnp.zeros((16,), jnp.float32)

    def zbody(j, _):
        hist[pl.ds(j * 16, 16)] = z
        return 0

    lax.fori_loop(0, nbins // 16, zbody, 0, unroll=8)


@functools.partial(
    pl.kernel,
    out_type=jax.ShapeDtypeStruct((_NW * _NCOPIES * _HBINS,), jnp.float32),
    mesh=_SC_MESH,
    compiler_params=pltpu.CompilerParams(needs_layout_passes=False),
    scratch_types=[
        pltpu.VMEM((_CHUNK,), jnp.int32),
        pltpu.VMEM((_HBINS,), jnp.float32),
        pltpu.VMEM((_HBINS,), jnp.float32),
        pltpu.VMEM((_HBINS,), jnp.float32),
        pltpu.VMEM((_HBINS,), jnp.float32),
    ],
)
def _sc_hist(adj_hbm, out_hbm, buf, h0, h1, h2, h3):
    wid = lax.axis_index("s") * 2 + lax.axis_index("c")
    base = wid * _EPW
    hists = (h0, h1, h2, h3)
    for h in hists:
        _zero_hist(h, _HBINS)
    ones = jnp.ones((16,), jnp.float32)

    def chunk_body(g, _):
        pltpu.sync_copy(adj_hbm.at[pl.ds(base + g * _CHUNK, _CHUNK)], buf)

        def ebody(j, _):
            for c in range(_NCOPIES):
                b = buf[pl.ds(j * (16 * _NCOPIES) + c * 16, 16)]
                binv = lax.shift_right_logical(b, _LOWBITS)
                plsc.addupdate_scatter(hists[c], [binv], ones)
            return 0

        lax.fori_loop(0, _CHUNK // (16 * _NCOPIES), ebody, 0, unroll=2)
        return 0

    lax.fori_loop(0, _NCHUNK, chunk_body, 0)
    for c in range(_NCOPIES):
        pltpu.sync_copy(
            hists[c],
            out_hbm.at[pl.ds((wid * _NCOPIES + c) * _HBINS, _HBINS)],
        )


def _suffix_counts(merged_f, nrows, ncols):
    rowtot = jnp.sum(merged_f, axis=1)
    a_r = lax.broadcasted_iota(jnp.int32, (nrows, nrows), 0)
    a_c = lax.broadcasted_iota(jnp.int32, (nrows, nrows), 1)
    strict_upper = (a_c > a_r).astype(jnp.float32)
    tail = lax.dot_general(
        strict_upper, rowtot, (((1,), (0,)), ((), ())), precision=_HIGHEST
    )
    u_k = lax.broadcasted_iota(jnp.int32, (ncols, ncols), 0)
    u_c = lax.broadcasted_iota(jnp.int32, (ncols, ncols), 1)
    upper = (u_k >= u_c).astype(jnp.float32)
    colsuf = lax.dot_general(
        merged_f, upper, (((1,), (0,)), ((), ())), precision=_HIGHEST
    )
    return tail[:, None] + colsuf


def _select_body(adj_ref, h_ref, kth_ref):
    merged = jnp.sum(h_ref[...], axis=0)
    s = _suffix_counts(merged, 128, 128)
    cond = (s >= float(K_KEEP)).astype(jnp.int32)
    bstar = jnp.sum(cond) - 1
    base = lax.shift_left(bstar, _LOWBITS)

    def search_body(_, carry):
        lo, hi = carry
        mid = lax.shift_right_logical(lo + hi, 1)
        t = base + mid
        cnt = jnp.float32(0.0)
        for i in range(_NBLK):
            blk = adj_ref[i * _ROWS_PER_BLK:(i + 1) * _ROWS_PER_BLK, :]
            bbits = lax.bitcast_convert_type(blk, jnp.int32)
            cnt += jnp.sum((bbits > t).astype(jnp.float32))
        take_low = cnt < float(K_KEEP)
        return (
            jnp.where(take_low, lo, mid + 1),
            jnp.where(take_low, mid, hi),
        )

    lo, _ = lax.fori_loop(
        0, _LOWBITS, search_body, (jnp.int32(0), jnp.int32((1 << _LOWBITS) - 1))
    )
    kth_ref[...] = jnp.zeros((8, 128), jnp.int32) + (base + lo)


def _tc_select(adj, hr):
    return pl.pallas_call(
        _select_body,
        out_shape=jax.ShapeDtypeStruct((8, 128), jnp.int32),
    )(adj, hr)


def _norm_body(adj_ref, e_ref, kth_ref, out_ref):
    i = pl.program_id(0)
    kth_bits = kth_ref[0, 0]
    bbits = lax.bitcast_convert_type(adj_ref[...], jnp.int32)
    kept = bbits > kth_bits
    e = jnp.where(kept, e_ref[...], 0.0)
    denom = jnp.sum(e, axis=1, keepdims=True)
    kept_any = denom >= 0.5
    p = jnp.where(kept_any, e / denom, 1.0 / NUM_NODE)
    cols = lax.broadcasted_iota(jnp.int32, (_ROWS_PER_BLK, NUM_NODE), 1)
    rows = lax.broadcasted_iota(jnp.int32, (_ROWS_PER_BLK, NUM_NODE), 0)
    rows = rows + i * _ROWS_PER_BLK
    out_ref[...] = jnp.where(rows == cols, 1.0, p)


def _tc_norm(adj, e, kth):
    return pl.pallas_call(
        _norm_body,
        grid=(_NBLK,),
        in_specs=[
            pl.BlockSpec((_ROWS_PER_BLK, NUM_NODE), lambda i: (i, 0)),
            pl.BlockSpec((_ROWS_PER_BLK, NUM_NODE), lambda i: (i, 0)),
            pl.BlockSpec((8, 128), lambda i: (0, 0)),
        ],
        out_specs=pl.BlockSpec((_ROWS_PER_BLK, NUM_NODE), lambda i: (i, 0)),
        out_shape=jax.ShapeDtypeStruct((NUM_NODE, NUM_NODE), jnp.float32),
    )(adj, e, kth)


def kernel(x, M1, M2):
    del x
    adj, e = _tc_adj(M1, M2)
    flat = lax.bitcast_convert_type(adj, jnp.int32).reshape(-1)
    h = _sc_hist(flat).reshape(_NW * _NCOPIES, 128, 128)
    kth = _tc_select(adj, h)
    return _tc_norm(adj, e, kth)


# speedup vs baseline: 1.3150x; 1.1472x over previous
import functools

import jax
import jax.numpy as jnp
from jax import lax
from jax.experimental import pallas as pl
from jax.experimental.pallas import tpu as pltpu
from jax.experimental.pallas import tpu_sc as plsc

NUM_NODE = 2048
RANK = 64
K_KEEP = int(0.01 * NUM_NODE * NUM_NODE)
_ROWS_PER_BLK = 128
_NBLK = NUM_NODE // _ROWS_PER_BLK

_NW = 32
_NELEM = NUM_NODE * NUM_NODE
_EPW = _NELEM // _NW
_CHUNK = 8192
_NCHUNK = _EPW // _CHUNK
_NCOPIES = 4
_HBITS = 14
_HBINS = 1 << _HBITS
_LOWBITS = 31 - _HBITS
_HIGHEST = jax.lax.Precision.HIGHEST


def _adj_body(m1_ref, m2_ref, adj_ref, e_ref):
    a = jnp.maximum(
        lax.dot_general(
            m1_ref[...], m2_ref[...],
            dimension_numbers=(((1,), (1,)), ((), ())),
            preferred_element_type=jnp.float32,
        ),
        0.0,
    )
    adj_ref[...] = a
    rm = jnp.max(a, axis=1, keepdims=True)
    e_ref[...] = jnp.exp(a - rm)


def _tc_adj(M1, M2):
    return pl.pallas_call(
        _adj_body,
        grid=(_NBLK,),
        in_specs=[
            pl.BlockSpec((_ROWS_PER_BLK, RANK), lambda i: (i, 0)),
            pl.BlockSpec((NUM_NODE, RANK), lambda i: (0, 0)),
        ],
        out_specs=[
            pl.BlockSpec((_ROWS_PER_BLK, NUM_NODE), lambda i: (i, 0)),
            pl.BlockSpec((_ROWS_PER_BLK, NUM_NODE), lambda i: (i, 0)),
        ],
        out_shape=[
            jax.ShapeDtypeStruct((NUM_NODE, NUM_NODE), jnp.float32),
            jax.ShapeDtypeStruct((NUM_NODE, NUM_NODE), jnp.float32),
        ],
    )(M1, M2)


_SC_MESH = plsc.VectorSubcoreMesh(core_axis_name="c", subcore_axis_name="s")


def _zero_hist(hist, nbins):
    z = jnp.zeros((16,), jnp.float32)

    def zbody(j, _):
        hist[pl.ds(j * 16, 16)] = z
        return 0

    lax.fori_loop(0, nbins // 16, zbody, 0, unroll=8)


@functools.partial(
    pl.kernel,
    out_type=jax.ShapeDtypeStruct((_NW * _NCOPIES * _HBINS,), jnp.float32),
    mesh=_SC_MESH,
    compiler_params=pltpu.CompilerParams(needs_layout_passes=False),
    scratch_types=[
        pltpu.VMEM((_CHUNK,), jnp.int32),
        pltpu.VMEM((_HBINS,), jnp.float32),
        pltpu.VMEM((_HBINS,), jnp.float32),
        pltpu.VMEM((_HBINS,), jnp.float32),
        pltpu.VMEM((_HBINS,), jnp.float32),
    ],
)
def _sc_hist(adj_hbm, out_hbm, buf, h0, h1, h2, h3):
    wid = lax.axis_index("s") * 2 + lax.axis_index("c")
    base = wid * _EPW
    hists = (h0, h1, h2, h3)
    for h in hists:
        _zero_hist(h, _HBINS)
    ones = jnp.ones((16,), jnp.float32)

    def chunk_body(g, _):
        pltpu.sync_copy(adj_hbm.at[pl.ds(base + g * _CHUNK, _CHUNK)], buf)

        @plsc.parallel_loop(0, _CHUNK // (16 * _NCOPIES), unroll=2)
        def ebody(j):
            for c in range(_NCOPIES):
                b = buf[pl.ds(j * (16 * _NCOPIES) + c * 16, 16)]
                binv = lax.shift_right_logical(b, _LOWBITS)
                plsc.addupdate_scatter(hists[c], [binv], ones)

        return 0

    lax.fori_loop(0, _NCHUNK, chunk_body, 0)
    for c in range(_NCOPIES):
        pltpu.sync_copy(
            hists[c],
            out_hbm.at[pl.ds((wid * _NCOPIES + c) * _HBINS, _HBINS)],
        )


def _suffix_counts(merged_f, nrows, ncols):
    rowtot = jnp.sum(merged_f, axis=1)
    a_r = lax.broadcasted_iota(jnp.int32, (nrows, nrows), 0)
    a_c = lax.broadcasted_iota(jnp.int32, (nrows, nrows), 1)
    strict_upper = (a_c > a_r).astype(jnp.float32)
    tail = lax.dot_general(
        strict_upper, rowtot, (((1,), (0,)), ((), ())), precision=_HIGHEST
    )
    u_k = lax.broadcasted_iota(jnp.int32, (ncols, ncols), 0)
    u_c = lax.broadcasted_iota(jnp.int32, (ncols, ncols), 1)
    upper = (u_k >= u_c).astype(jnp.float32)
    colsuf = lax.dot_general(
        merged_f, upper, (((1,), (0,)), ((), ())), precision=_HIGHEST
    )
    return tail[:, None] + colsuf


def _select_body(adj_ref, h_ref, kth_ref):
    merged = jnp.sum(h_ref[...], axis=0)
    s = _suffix_counts(merged, 128, 128)
    cond = (s >= float(K_KEEP)).astype(jnp.int32)
    bstar = jnp.sum(cond) - 1
    base = lax.shift_left(bstar, _LOWBITS)

    def search_body(_, carry):
        lo, hi = carry
        mid = lax.shift_right_logical(lo + hi, 1)
        t = base + mid
        cnt = jnp.float32(0.0)
        for i in range(_NBLK):
            blk = adj_ref[i * _ROWS_PER_BLK:(i + 1) * _ROWS_PER_BLK, :]
            bbits = lax.bitcast_convert_type(blk, jnp.int32)
            cnt += jnp.sum((bbits > t).astype(jnp.float32))
        take_low = cnt < float(K_KEEP)
        return (
            jnp.where(take_low, lo, mid + 1),
            jnp.where(take_low, mid, hi),
        )

    lo, _ = lax.fori_loop(
        0, _LOWBITS, search_body, (jnp.int32(0), jnp.int32((1 << _LOWBITS) - 1))
    )
    kth_ref[...] = jnp.zeros((8, 128), jnp.int32) + (base + lo)


def _tc_select(adj, hr):
    return pl.pallas_call(
        _select_body,
        out_shape=jax.ShapeDtypeStruct((8, 128), jnp.int32),
    )(adj, hr)


def _norm_body(adj_ref, e_ref, kth_ref, out_ref):
    i = pl.program_id(0)
    kth_bits = kth_ref[0, 0]
    bbits = lax.bitcast_convert_type(adj_ref[...], jnp.int32)
    kept = bbits > kth_bits
    e = jnp.where(kept, e_ref[...], 0.0)
    denom = jnp.sum(e, axis=1, keepdims=True)
    kept_any = denom >= 0.5
    p = jnp.where(kept_any, e / denom, 1.0 / NUM_NODE)
    cols = lax.broadcasted_iota(jnp.int32, (_ROWS_PER_BLK, NUM_NODE), 1)
    rows = lax.broadcasted_iota(jnp.int32, (_ROWS_PER_BLK, NUM_NODE), 0)
    rows = rows + i * _ROWS_PER_BLK
    out_ref[...] = jnp.where(rows == cols, 1.0, p)


def _tc_norm(adj, e, kth):
    return pl.pallas_call(
        _norm_body,
        grid=(_NBLK,),
        in_specs=[
            pl.BlockSpec((_ROWS_PER_BLK, NUM_NODE), lambda i: (i, 0)),
            pl.BlockSpec((_ROWS_PER_BLK, NUM_NODE), lambda i: (i, 0)),
            pl.BlockSpec((8, 128), lambda i: (0, 0)),
        ],
        out_specs=pl.BlockSpec((_ROWS_PER_BLK, NUM_NODE), lambda i: (i, 0)),
        out_shape=jax.ShapeDtypeStruct((NUM_NODE, NUM_NODE), jnp.float32),
    )(adj, e, kth)


def kernel(x, M1, M2):
    del x
    adj, e = _tc_adj(M1, M2)
    flat = lax.bitcast_convert_type(adj, jnp.int32).reshape(-1)
    h = _sc_hist(flat).reshape(_NW * _NCOPIES, 128, 128)
    kth = _tc_select(adj, h)
    return _tc_norm(adj, e, kth)


# speedup vs baseline: 1.4058x; 1.0690x over previous
import functools

import jax
import jax.numpy as jnp
from jax import lax
from jax.experimental import pallas as pl
from jax.experimental.pallas import tpu as pltpu
from jax.experimental.pallas import tpu_sc as plsc

NUM_NODE = 2048
RANK = 64
K_KEEP = int(0.01 * NUM_NODE * NUM_NODE)
_ROWS_PER_BLK = 128
_NBLK = NUM_NODE // _ROWS_PER_BLK

_NW = 32
_NELEM = NUM_NODE * NUM_NODE
_EPW = _NELEM // _NW
_CHUNK = 16384
_NCHUNK = _EPW // _CHUNK
_NCOPIES = 4
_HBITS = 14
_HBINS = 1 << _HBITS
_LOWBITS = 31 - _HBITS
_HIGHEST = jax.lax.Precision.HIGHEST


def _adj_body(m1_ref, m2_ref, adj_ref):
    adj_ref[...] = jnp.maximum(
        lax.dot_general(
            m1_ref[...], m2_ref[...],
            dimension_numbers=(((1,), (1,)), ((), ())),
            preferred_element_type=jnp.float32,
        ),
        0.0,
    )


def _tc_adj(M1, M2):
    return pl.pallas_call(
        _adj_body,
        grid=(_NBLK,),
        in_specs=[
            pl.BlockSpec((_ROWS_PER_BLK, RANK), lambda i: (i, 0)),
            pl.BlockSpec((NUM_NODE, RANK), lambda i: (0, 0)),
        ],
        out_specs=pl.BlockSpec((_ROWS_PER_BLK, NUM_NODE), lambda i: (i, 0)),
        out_shape=jax.ShapeDtypeStruct((NUM_NODE, NUM_NODE), jnp.float32),
    )(M1, M2)


_SC_MESH = plsc.VectorSubcoreMesh(core_axis_name="c", subcore_axis_name="s")


def _zero_hist(hist, nbins):
    z = jnp.zeros((16,), jnp.float32)

    def zbody(j, _):
        hist[pl.ds(j * 16, 16)] = z
        return 0

    lax.fori_loop(0, nbins // 16, zbody, 0, unroll=8)


@functools.partial(
    pl.kernel,
    out_type=jax.ShapeDtypeStruct((_NW * _HBINS,), jnp.float32),
    mesh=_SC_MESH,
    compiler_params=pltpu.CompilerParams(needs_layout_passes=False),
    scratch_types=[
        pltpu.VMEM((_CHUNK,), jnp.int32),
        pltpu.VMEM((_CHUNK,), jnp.int32),
        pltpu.VMEM((_HBINS,), jnp.float32),
        pltpu.VMEM((_HBINS,), jnp.float32),
        pltpu.VMEM((_HBINS,), jnp.float32),
        pltpu.VMEM((_HBINS,), jnp.float32),
        pltpu.SemaphoreType.DMA,
        pltpu.SemaphoreType.DMA,
    ],
)
def _sc_hist(adj_hbm, out_hbm, bufa, bufb, h0, h1, h2, h3, sema, semb):
    wid = lax.axis_index("s") * 2 + lax.axis_index("c")
    base = wid * _EPW
    bufs = (bufa, bufb)
    sems = (sema, semb)
    hists = (h0, h1, h2, h3)
    for h in hists:
        _zero_hist(h, _HBINS)
    ones = jnp.ones((16,), jnp.float32)

    def _start(g, b):
        pltpu.async_copy(
            adj_hbm.at[pl.ds(base + g * _CHUNK, _CHUNK)], bufs[b], sems[b]
        )

    def _process(buf):
        @plsc.parallel_loop(0, _CHUNK // (16 * _NCOPIES), unroll=2)
        def ebody(j):
            for c in range(_NCOPIES):
                b = buf[pl.ds(j * (16 * _NCOPIES) + c * 16, 16)]
                binv = lax.shift_right_logical(b, _LOWBITS)
                plsc.addupdate_scatter(hists[c], [binv], ones)

    _start(0, 0)

    def pair_body(gg, _):
        for b in range(2):
            g = gg * 2 + b

            @pl.when(g + 1 < _NCHUNK)
            def _():
                _start(g + 1, (b + 1) % 2)

            pltpu.make_async_copy(
                adj_hbm.at[pl.ds(base + g * _CHUNK, _CHUNK)],
                bufs[b], sems[b],
            ).wait()
            _process(bufs[b])
        return 0

    lax.fori_loop(0, _NCHUNK // 2, pair_body, 0)

    def merge_body(j, _):
        s = pl.ds(j * 16, 16)
        h0[s] = h0[s] + h1[s] + h2[s] + h3[s]
        return 0

    lax.fori_loop(0, _HBINS // 16, merge_body, 0, unroll=8)
    pltpu.sync_copy(h0, out_hbm.at[pl.ds(wid * _HBINS, _HBINS)])


def _suffix_counts(merged_f, nrows, ncols):
    rowtot = jnp.sum(merged_f, axis=1)
    a_r = lax.broadcasted_iota(jnp.int32, (nrows, nrows), 0)
    a_c = lax.broadcasted_iota(jnp.int32, (nrows, nrows), 1)
    strict_upper = (a_c > a_r).astype(jnp.float32)
    tail = lax.dot_general(
        strict_upper, rowtot, (((1,), (0,)), ((), ())), precision=_HIGHEST
    )
    u_k = lax.broadcasted_iota(jnp.int32, (ncols, ncols), 0)
    u_c = lax.broadcasted_iota(jnp.int32, (ncols, ncols), 1)
    upper = (u_k >= u_c).astype(jnp.float32)
    colsuf = lax.dot_general(
        merged_f, upper, (((1,), (0,)), ((), ())), precision=_HIGHEST
    )
    return tail[:, None] + colsuf


def _select_body(adj_ref, h_ref, kth_ref):
    merged = jnp.sum(h_ref[...], axis=0)
    s = _suffix_counts(merged, 128, 128)
    cond = (s >= float(K_KEEP)).astype(jnp.int32)
    bstar = jnp.sum(cond) - 1
    base = lax.shift_left(bstar, _LOWBITS)

    def search_body(_, carry):
        lo, hi = carry
        mid = lax.shift_right_logical(lo + hi, 1)
        t = base + mid
        cnt = jnp.float32(0.0)
        for i in range(_NBLK):
            blk = adj_ref[i * _ROWS_PER_BLK:(i + 1) * _ROWS_PER_BLK, :]
            bbits = lax.bitcast_convert_type(blk, jnp.int32)
            cnt += jnp.sum((bbits > t).astype(jnp.float32))
        take_low = cnt < float(K_KEEP)
        return (
            jnp.where(take_low, lo, mid + 1),
            jnp.where(take_low, mid, hi),
        )

    lo, _ = lax.fori_loop(
        0, _LOWBITS, search_body, (jnp.int32(0), jnp.int32((1 << _LOWBITS) - 1))
    )
    kth_ref[...] = jnp.zeros((8, 128), jnp.int32) + (base + lo)


def _tc_select(adj, hr):
    return pl.pallas_call(
        _select_body,
        out_shape=jax.ShapeDtypeStruct((8, 128), jnp.int32),
    )(adj, hr)


def _norm_body(adj_ref, kth_ref, out_ref):
    i = pl.program_id(0)
    kth_bits = kth_ref[0, 0]
    a = adj_ref[...]
    bbits = lax.bitcast_convert_type(a, jnp.int32)
    kept = bbits > kth_bits
    rm = jnp.max(a, axis=1, keepdims=True)
    e = jnp.where(kept, jnp.exp(a - rm), 0.0)
    denom = jnp.sum(e, axis=1, keepdims=True)
    kept_any = denom >= 0.5
    p = jnp.where(kept_any, e / denom, 1.0 / NUM_NODE)
    cols = lax.broadcasted_iota(jnp.int32, (_ROWS_PER_BLK, NUM_NODE), 1)
    rows = lax.broadcasted_iota(jnp.int32, (_ROWS_PER_BLK, NUM_NODE), 0)
    rows = rows + i * _ROWS_PER_BLK
    out_ref[...] = jnp.where(rows == cols, 1.0, p)


def _tc_norm(adj, kth):
    return pl.pallas_call(
        _norm_body,
        grid=(_NBLK,),
        in_specs=[
            pl.BlockSpec((_ROWS_PER_BLK, NUM_NODE), lambda i: (i, 0)),
            pl.BlockSpec((8, 128), lambda i: (0, 0)),
        ],
        out_specs=pl.BlockSpec((_ROWS_PER_BLK, NUM_NODE), lambda i: (i, 0)),
        out_shape=jax.ShapeDtypeStruct((NUM_NODE, NUM_NODE), jnp.float32),
    )(adj, kth)


def kernel(x, M1, M2):
    del x
    adj = _tc_adj(M1, M2)
    flat = lax.bitcast_convert_type(adj, jnp.int32).reshape(-1)
    h = _sc_hist(flat).reshape(_NW, 128, 128)
    kth = _tc_select(adj, h)
    return _tc_norm(adj, kth)


# speedup vs baseline: 2.1735x; 1.5461x over previous
import functools

import jax
import jax.numpy as jnp
from jax import lax
from jax.experimental import pallas as pl
from jax.experimental.pallas import tpu as pltpu
from jax.experimental.pallas import tpu_sc as plsc

NUM_NODE = 2048
RANK = 64
K_KEEP = int(0.01 * NUM_NODE * NUM_NODE)
_ROWS_PER_BLK = 128
_NBLK = NUM_NODE // _ROWS_PER_BLK

_NW = 32
_NELEM = NUM_NODE * NUM_NODE
_EPW = _NELEM // _NW
_CHUNK = 16384
_NCHUNK = _EPW // _CHUNK
_NCOPIES = 4
_HBITS = 14
_HBINS = 1 << _HBITS
_LOWBITS = 31 - _HBITS
_HIGHEST = jax.lax.Precision.HIGHEST


def _adj_body(m1_ref, m2_ref, adj_ref):
    adj_ref[...] = jnp.maximum(
        lax.dot_general(
            m1_ref[...], m2_ref[...],
            dimension_numbers=(((1,), (1,)), ((), ())),
            preferred_element_type=jnp.float32,
        ),
        0.0,
    )


def _tc_adj(M1, M2):
    return pl.pallas_call(
        _adj_body,
        grid=(_NBLK,),
        in_specs=[
            pl.BlockSpec((_ROWS_PER_BLK, RANK), lambda i: (i, 0)),
            pl.BlockSpec((NUM_NODE, RANK), lambda i: (0, 0)),
        ],
        out_specs=pl.BlockSpec((_ROWS_PER_BLK, NUM_NODE), lambda i: (i, 0)),
        out_shape=jax.ShapeDtypeStruct((NUM_NODE, NUM_NODE), jnp.float32),
    )(M1, M2)


_SC_MESH = plsc.VectorSubcoreMesh(core_axis_name="c", subcore_axis_name="s")


def _zero_hist(hist, nbins):
    z = jnp.zeros((16,), jnp.float32)

    def zbody(j, _):
        hist[pl.ds(j * 16, 16)] = z
        return 0

    lax.fori_loop(0, nbins // 16, zbody, 0, unroll=8)


_ROWS_PER_W = NUM_NODE // _NW
_ROWS_PER_CHUNK = _CHUNK // NUM_NODE


@functools.partial(
    pl.kernel,
    out_type=jax.ShapeDtypeStruct((_NW * _HBINS,), jnp.float32),
    mesh=_SC_MESH,
    compiler_params=pltpu.CompilerParams(needs_layout_passes=False),
    scratch_types=[
        pltpu.VMEM((_ROWS_PER_CHUNK, NUM_NODE), jnp.int32),
        pltpu.VMEM((_ROWS_PER_CHUNK, NUM_NODE), jnp.int32),
        pltpu.VMEM((_HBINS,), jnp.float32),
        pltpu.VMEM((_HBINS,), jnp.float32),
        pltpu.VMEM((_HBINS,), jnp.float32),
        pltpu.VMEM((_HBINS,), jnp.float32),
        pltpu.SemaphoreType.DMA,
        pltpu.SemaphoreType.DMA,
    ],
)
def _sc_hist(adj_hbm, out_hbm, bufa, bufb, h0, h1, h2, h3, sema, semb):
    wid = lax.axis_index("s") * 2 + lax.axis_index("c")
    base_row = wid * _ROWS_PER_W
    bufs = (bufa, bufb)
    sems = (sema, semb)
    hists = (h0, h1, h2, h3)
    for h in hists:
        _zero_hist(h, _HBINS)
    ones = jnp.ones((16,), jnp.float32)
    zero16 = jnp.zeros((16,), jnp.int32)

    def _start(g, b):
        pltpu.async_copy(
            adj_hbm.at[pl.ds(base_row + g * _ROWS_PER_CHUNK, _ROWS_PER_CHUNK)],
            bufs[b], sems[b],
        )

    def _process(buf):
        for r in range(_ROWS_PER_CHUNK):
            @plsc.parallel_loop(0, NUM_NODE // (16 * _NCOPIES), unroll=2)
            def ebody(j):
                for c in range(_NCOPIES):
                    b = buf[r, pl.ds(j * (16 * _NCOPIES) + c * 16, 16)]
                    binv = lax.shift_right_logical(b, _LOWBITS)
                    plsc.addupdate_scatter(
                        hists[c], [binv], ones, mask=b != zero16
                    )

    _start(0, 0)

    def pair_body(gg, _):
        for b in range(2):
            g = gg * 2 + b

            @pl.when(g + 1 < _NCHUNK)
            def _():
                _start(g + 1, (b + 1) % 2)

            pltpu.make_async_copy(
                adj_hbm.at[
                    pl.ds(base_row + g * _ROWS_PER_CHUNK, _ROWS_PER_CHUNK)
                ],
                bufs[b], sems[b],
            ).wait()
            _process(bufs[b])
        return 0

    lax.fori_loop(0, _NCHUNK // 2, pair_body, 0)

    def merge_body(j, _):
        s = pl.ds(j * 16, 16)
        h0[s] = h0[s] + h1[s] + h2[s] + h3[s]
        return 0

    lax.fori_loop(0, _HBINS // 16, merge_body, 0, unroll=8)
    pltpu.sync_copy(h0, out_hbm.at[pl.ds(wid * _HBINS, _HBINS)])


def _suffix_counts(merged_f, nrows, ncols):
    rowtot = jnp.sum(merged_f, axis=1)
    a_r = lax.broadcasted_iota(jnp.int32, (nrows, nrows), 0)
    a_c = lax.broadcasted_iota(jnp.int32, (nrows, nrows), 1)
    strict_upper = (a_c > a_r).astype(jnp.float32)
    tail = lax.dot_general(
        strict_upper, rowtot, (((1,), (0,)), ((), ())), precision=_HIGHEST
    )
    u_k = lax.broadcasted_iota(jnp.int32, (ncols, ncols), 0)
    u_c = lax.broadcasted_iota(jnp.int32, (ncols, ncols), 1)
    upper = (u_k >= u_c).astype(jnp.float32)
    colsuf = lax.dot_general(
        merged_f, upper, (((1,), (0,)), ((), ())), precision=_HIGHEST
    )
    return tail[:, None] + colsuf


def _select_body(adj_ref, h_ref, kth_ref):
    merged = jnp.sum(h_ref[...], axis=0)
    r_i = lax.broadcasted_iota(jnp.int32, (128, 128), 0)
    c_i = lax.broadcasted_iota(jnp.int32, (128, 128), 1)
    nzeros = float(_NELEM) - jnp.sum(merged)
    merged = jnp.where((r_i == 0) & (c_i == 0), merged + nzeros, merged)
    s = _suffix_counts(merged, 128, 128)
    cond = (s >= float(K_KEEP)).astype(jnp.int32)
    bstar = jnp.sum(cond) - 1
    base = lax.shift_left(bstar, _LOWBITS)

    def search_body(_, carry):
        lo, hi = carry
        mid = lax.shift_right_logical(lo + hi, 1)
        t = base + mid
        cnt = jnp.float32(0.0)
        for i in range(_NBLK):
            blk = adj_ref[i * _ROWS_PER_BLK:(i + 1) * _ROWS_PER_BLK, :]
            bbits = lax.bitcast_convert_type(blk, jnp.int32)
            cnt += jnp.sum((bbits > t).astype(jnp.float32))
        take_low = cnt < float(K_KEEP)
        return (
            jnp.where(take_low, lo, mid + 1),
            jnp.where(take_low, mid, hi),
        )

    lo, _ = lax.fori_loop(
        0, _LOWBITS, search_body, (jnp.int32(0), jnp.int32((1 << _LOWBITS) - 1))
    )
    kth_ref[...] = jnp.zeros((8, 128), jnp.int32) + (base + lo)


def _tc_select(adj, hr):
    return pl.pallas_call(
        _select_body,
        out_shape=jax.ShapeDtypeStruct((8, 128), jnp.int32),
    )(adj, hr)


def _norm_body(adj_ref, kth_ref, out_ref):
    i = pl.program_id(0)
    kth_bits = kth_ref[0, 0]
    a = adj_ref[...]
    bbits = lax.bitcast_convert_type(a, jnp.int32)
    kept = bbits > kth_bits
    rm = jnp.max(a, axis=1, keepdims=True)
    e = jnp.where(kept, jnp.exp(a - rm), 0.0)
    denom = jnp.sum(e, axis=1, keepdims=True)
    kept_any = denom >= 0.5
    p = jnp.where(kept_any, e / denom, 1.0 / NUM_NODE)
    cols = lax.broadcasted_iota(jnp.int32, (_ROWS_PER_BLK, NUM_NODE), 1)
    rows = lax.broadcasted_iota(jnp.int32, (_ROWS_PER_BLK, NUM_NODE), 0)
    rows = rows + i * _ROWS_PER_BLK
    out_ref[...] = jnp.where(rows == cols, 1.0, p)


def _tc_norm(adj, kth):
    return pl.pallas_call(
        _norm_body,
        grid=(_NBLK,),
        in_specs=[
            pl.BlockSpec((_ROWS_PER_BLK, NUM_NODE), lambda i: (i, 0)),
            pl.BlockSpec((8, 128), lambda i: (0, 0)),
        ],
        out_specs=pl.BlockSpec((_ROWS_PER_BLK, NUM_NODE), lambda i: (i, 0)),
        out_shape=jax.ShapeDtypeStruct((NUM_NODE, NUM_NODE), jnp.float32),
    )(adj, kth)


def kernel(x, M1, M2):
    del x
    adj = _tc_adj(M1, M2)
    bits2d = lax.bitcast_convert_type(adj, jnp.int32)
    h = _sc_hist(bits2d).reshape(_NW, 128, 128)
    kth = _tc_select(adj, h)
    return _tc_norm(adj, kth)


# speedup vs baseline: 2.3596x; 1.0856x over previous
import functools

import jax
import jax.numpy as jnp
from jax import lax
from jax.experimental import pallas as pl
from jax.experimental.pallas import tpu as pltpu
from jax.experimental.pallas import tpu_sc as plsc

NUM_NODE = 2048
RANK = 64
K_KEEP = int(0.01 * NUM_NODE * NUM_NODE)
_ROWS_PER_BLK = 128
_NBLK = NUM_NODE // _ROWS_PER_BLK

_NW = 32
_NELEM = NUM_NODE * NUM_NODE
_EPW = _NELEM // _NW
_CHUNK = 16384
_NCHUNK = _EPW // _CHUNK
_NCOPIES = 4
_HBITS = 14
_HBINS = 1 << _HBITS
_LOWBITS = 31 - _HBITS
_HIGHEST = jax.lax.Precision.HIGHEST


def _adj_body(m1_ref, m2_ref, adj_ref):
    adj_ref[...] = jnp.maximum(
        lax.dot_general(
            m1_ref[...], m2_ref[...],
            dimension_numbers=(((1,), (1,)), ((), ())),
            preferred_element_type=jnp.float32,
        ),
        0.0,
    )


def _tc_adj(M1, M2):
    return pl.pallas_call(
        _adj_body,
        grid=(_NBLK,),
        in_specs=[
            pl.BlockSpec((_ROWS_PER_BLK, RANK), lambda i: (i, 0)),
            pl.BlockSpec((NUM_NODE, RANK), lambda i: (0, 0)),
        ],
        out_specs=pl.BlockSpec((_ROWS_PER_BLK, NUM_NODE), lambda i: (i, 0)),
        out_shape=jax.ShapeDtypeStruct((NUM_NODE, NUM_NODE), jnp.float32),
    )(M1, M2)


_SC_MESH = plsc.VectorSubcoreMesh(core_axis_name="c", subcore_axis_name="s")


def _zero_hist(hist, nbins):
    z = jnp.zeros((16,), jnp.float32)

    def zbody(j, _):
        hist[pl.ds(j * 16, 16)] = z
        return 0

    lax.fori_loop(0, nbins // 16, zbody, 0, unroll=8)


_ROWS_PER_W = NUM_NODE // _NW
_ROWS_PER_CHUNK = _CHUNK // NUM_NODE


@functools.partial(
    pl.kernel,
    out_type=jax.ShapeDtypeStruct((_NW * _HBINS,), jnp.float32),
    mesh=_SC_MESH,
    compiler_params=pltpu.CompilerParams(needs_layout_passes=False),
    scratch_types=[
        pltpu.VMEM((_ROWS_PER_CHUNK, NUM_NODE), jnp.int32),
        pltpu.VMEM((_ROWS_PER_CHUNK, NUM_NODE), jnp.int32),
        pltpu.VMEM((_HBINS,), jnp.float32),
        pltpu.VMEM((_HBINS,), jnp.float32),
        pltpu.VMEM((_HBINS,), jnp.float32),
        pltpu.VMEM((_HBINS,), jnp.float32),
        pltpu.SemaphoreType.DMA,
        pltpu.SemaphoreType.DMA,
    ],
)
def _sc_hist(adj_hbm, out_hbm, bufa, bufb, h0, h1, h2, h3, sema, semb):
    wid = lax.axis_index("s") * 2 + lax.axis_index("c")
    base_row = wid * _ROWS_PER_W
    bufs = (bufa, bufb)
    sems = (sema, semb)
    hists = (h0, h1, h2, h3)
    for h in hists:
        _zero_hist(h, _HBINS)
    ones = jnp.ones((16,), jnp.float32)
    zero16 = jnp.zeros((16,), jnp.int32)

    def _start(g, b):
        pltpu.async_copy(
            adj_hbm.at[pl.ds(base_row + g * _ROWS_PER_CHUNK, _ROWS_PER_CHUNK)],
            bufs[b], sems[b],
        )

    def _process(buf):
        for r in range(_ROWS_PER_CHUNK):
            @plsc.parallel_loop(0, NUM_NODE // (16 * _NCOPIES), unroll=2)
            def ebody(j):
                for c in range(_NCOPIES):
                    b = buf[r, pl.ds(j * (16 * _NCOPIES) + c * 16, 16)]
                    binv = lax.shift_right_logical(b, _LOWBITS)
                    plsc.addupdate_scatter(
                        hists[c], [binv], ones, mask=b != zero16
                    )

    _start(0, 0)

    def pair_body(gg, _):
        for b in range(2):
            g = gg * 2 + b

            @pl.when(g + 1 < _NCHUNK)
            def _():
                _start(g + 1, (b + 1) % 2)

            pltpu.make_async_copy(
                adj_hbm.at[
                    pl.ds(base_row + g * _ROWS_PER_CHUNK, _ROWS_PER_CHUNK)
                ],
                bufs[b], sems[b],
            ).wait()
            _process(bufs[b])
        return 0

    lax.fori_loop(0, _NCHUNK // 2, pair_body, 0)

    def merge_body(j, _):
        s = pl.ds(j * 16, 16)
        h0[s] = h0[s] + h1[s] + h2[s] + h3[s]
        return 0

    lax.fori_loop(0, _HBINS // 16, merge_body, 0, unroll=8)
    pltpu.sync_copy(h0, out_hbm.at[pl.ds(wid * _HBINS, _HBINS)])


def _suffix_counts(merged_f, nrows, ncols):
    rowtot = jnp.sum(merged_f, axis=1)
    a_r = lax.broadcasted_iota(jnp.int32, (nrows, nrows), 0)
    a_c = lax.broadcasted_iota(jnp.int32, (nrows, nrows), 1)
    strict_upper = (a_c > a_r).astype(jnp.float32)
    tail = lax.dot_general(
        strict_upper, rowtot, (((1,), (0,)), ((), ())), precision=_HIGHEST
    )
    u_k = lax.broadcasted_iota(jnp.int32, (ncols, ncols), 0)
    u_c = lax.broadcasted_iota(jnp.int32, (ncols, ncols), 1)
    upper = (u_k >= u_c).astype(jnp.float32)
    colsuf = lax.dot_general(
        merged_f, upper, (((1,), (0,)), ((), ())), precision=_HIGHEST
    )
    return tail[:, None] + colsuf


def _select_norm_body(adj_ref, h_ref, out_ref):
    merged = jnp.sum(h_ref[...], axis=0)
    r_i = lax.broadcasted_iota(jnp.int32, (128, 128), 0)
    c_i = lax.broadcasted_iota(jnp.int32, (128, 128), 1)
    nzeros = float(_NELEM) - jnp.sum(merged)
    merged = jnp.where((r_i == 0) & (c_i == 0), merged + nzeros, merged)
    s = _suffix_counts(merged, 128, 128)
    cond = (s >= float(K_KEEP)).astype(jnp.int32)
    bstar = jnp.sum(cond) - 1
    base = lax.shift_left(bstar, _LOWBITS)

    def search_body(_, carry):
        lo, hi = carry
        mid = lax.shift_right_logical(lo + hi, 1)
        t = base + mid
        cnt = jnp.float32(0.0)
        for i in range(_NBLK):
            blk = adj_ref[i * _ROWS_PER_BLK:(i + 1) * _ROWS_PER_BLK, :]
            bbits = lax.bitcast_convert_type(blk, jnp.int32)
            cnt += jnp.sum((bbits > t).astype(jnp.float32))
        take_low = cnt < float(K_KEEP)
        return (
            jnp.where(take_low, lo, mid + 1),
            jnp.where(take_low, mid, hi),
        )

    lo, _ = lax.fori_loop(
        0, _LOWBITS, search_body, (jnp.int32(0), jnp.int32((1 << _LOWBITS) - 1))
    )
    kth_bits = base + lo

    cols = lax.broadcasted_iota(jnp.int32, (_ROWS_PER_BLK, NUM_NODE), 1)
    rows0 = lax.broadcasted_iota(jnp.int32, (_ROWS_PER_BLK, NUM_NODE), 0)
    for i in range(_NBLK):
        a = adj_ref[i * _ROWS_PER_BLK:(i + 1) * _ROWS_PER_BLK, :]
        bbits = lax.bitcast_convert_type(a, jnp.int32)
        kept = bbits > kth_bits
        rm = jnp.max(a, axis=1, keepdims=True)
        e = jnp.where(kept, jnp.exp(a - rm), 0.0)
        denom = jnp.sum(e, axis=1, keepdims=True)
        kept_any = denom >= 0.5
        p = jnp.where(kept_any, e / denom, 1.0 / NUM_NODE)
        rows = rows0 + i * _ROWS_PER_BLK
        out_ref[i * _ROWS_PER_BLK:(i + 1) * _ROWS_PER_BLK, :] = jnp.where(
            rows == cols, 1.0, p
        )


def _tc_select_norm(adj, hr):
    return pl.pallas_call(
        _select_norm_body,
        out_shape=jax.ShapeDtypeStruct((NUM_NODE, NUM_NODE), jnp.float32),
    )(adj, hr)


def kernel(x, M1, M2):
    del x
    adj = _tc_adj(M1, M2)
    bits2d = lax.bitcast_convert_type(adj, jnp.int32)
    h = _sc_hist(bits2d).reshape(_NW, 128, 128)
    return _tc_select_norm(adj, h)
